# Initial kernel scaffold; baseline (speedup 1.0000x reference)
import functools

import jax
import jax.numpy as jnp
from jax import lax
from jax.experimental import pallas as pl
from jax.experimental.pallas import tpu as pltpu
from jax.experimental.pallas import tpu_sc as plsc

NHID = 128
HALF = 64
NPER = 16384
E = 262144
NC = 2
NS = 16
CH = 128
EDGES_PER_TILE = E // NS
NCHUNK = EDGES_PER_TILE // CH
ROWS_PER_TILE = NPER // NS


def _sc_segsum_body(table, src_hbm, dst_hbm, out, sidx, didx, rows, acc, sem):
    c = lax.axis_index("c")
    s = lax.axis_index("s")

    zero = jnp.zeros((16,), jnp.float32)

    def zrow(i, carry):
        for j in range(HALF // 16):
            rows[i, pl.ds(j * 16, 16)] = zero
        return carry

    lax.fori_loop(0, CH, zrow, 0)
    for k in range(ROWS_PER_TILE // CH):
        pltpu.sync_copy(rows, acc.at[pl.ds(s * ROWS_PER_TILE + k * CH, CH)])
    plsc.subcore_barrier()

    coff = c * NPER

    def chunk(k, carry):
        base = s * EDGES_PER_TILE + k * CH
        pltpu.sync_copy(src_hbm.at[pl.ds(base, CH)], sidx)
        pltpu.sync_copy(dst_hbm.at[pl.ds(base, CH)], didx)
        for j in range(CH // 16):
            sidx[pl.ds(j * 16, 16)] = sidx[pl.ds(j * 16, 16)] + coff
        pltpu.async_copy(table.at[sidx], rows, sem).wait()
        pltpu.sync_copy(rows, acc.at[didx], add=True)
        return carry

    lax.fori_loop(0, NCHUNK, chunk, 0)
    plsc.subcore_barrier()
    pltpu.sync_copy(acc.at[pl.ds(s * ROWS_PER_TILE, ROWS_PER_TILE)],
                    out.at[pl.ds(coff + s * ROWS_PER_TILE, ROWS_PER_TILE)])


_sc_segsum = functools.partial(
    pl.kernel,
    mesh=plsc.VectorSubcoreMesh(core_axis_name="c", subcore_axis_name="s"),
    out_type=jax.ShapeDtypeStruct((NC * NPER, HALF), jnp.float32),
    scratch_types=[
        pltpu.VMEM((CH,), jnp.int32),
        pltpu.VMEM((CH,), jnp.int32),
        pltpu.VMEM((CH, HALF), jnp.float32),
        pltpu.VMEM_SHARED((NPER, HALF), jnp.float32),
        pltpu.SemaphoreType.DMA,
    ],
)(_sc_segsum_body)


def _mlp_body(a_ref, xres_ref, w0_ref, g0_ref, b0_ref, w1_ref, g1_ref,
              b1_ref, eps_ref, out_ref):
    a = a_ref[...]
    h = lax.dot_general(a, w0_ref[...], (((1,), (1,)), ((), ())),
                        preferred_element_type=jnp.float32)
    m = jnp.mean(h, axis=0, keepdims=True)
    hm = h - m
    v = jnp.mean(hm * hm, axis=0, keepdims=True)
    x1 = jnp.maximum(hm * lax.rsqrt(v + 1e-5) * g0_ref[...] + b0_ref[...], 0.0)
    h2 = lax.dot_general(x1, w1_ref[...], (((1,), (1,)), ((), ())),
                         preferred_element_type=jnp.float32)
    m2 = jnp.mean(h2, axis=0, keepdims=True)
    hm2 = h2 - m2
    v2 = jnp.mean(hm2 * hm2, axis=0, keepdims=True)
    y = jnp.maximum(hm2 * lax.rsqrt(v2 + 1e-5) * g1_ref[...] + b1_ref[...], 0.0)
    out_ref[...] = (1.0 + eps_ref[0, 0]) * xres_ref[...] + y


_mlp = pl.pallas_call(
    _mlp_body,
    out_shape=jax.ShapeDtypeStruct((NPER, NHID), jnp.float32),
    in_specs=[pl.BlockSpec(memory_space=pltpu.VMEM)] * 8
             + [pl.BlockSpec(memory_space=pltpu.SMEM)],
)


def _split(x):
    n = x.shape[0]
    return x.reshape(n, 2, HALF).transpose(1, 0, 2).reshape(2 * n, HALF)


def _merge(t):
    n = t.shape[0] // 2
    return t.reshape(2, n, HALF).transpose(1, 0, 2).reshape(n, NHID)


def _phase(x_src, x_res, src_idx, dst_idx, W0, g0, b0, W1, g1, b1, eps):
    agg = _sc_segsum(_split(x_src), src_idx, dst_idx)
    return _mlp(_merge(agg), x_res,
                W0, g0.reshape(1, NHID), b0.reshape(1, NHID),
                W1, g1.reshape(1, NHID), b1.reshape(1, NHID),
                eps.reshape(1, 1))


def kernel(xs, k_batch, bipartites, bw_W0, bw_g0, bw_b0, bw_W1, bw_g1, bw_b1,
           fw_W0, fw_g0, fw_b0, fw_W1, fw_g1, fw_b1, bw_eps, fw_eps):
    L0 = xs[0:NPER]
    L1 = xs[NPER:2 * NPER]
    L2 = xs[2 * NPER:3 * NPER]
    ei0 = bipartites[0]
    ei1 = bipartites[1]

    newL1 = _phase(L2, L1, ei1[1], ei1[0],
                   bw_W0[1], bw_g0[1], bw_b0[1], bw_W1[1], bw_g1[1], bw_b1[1],
                   bw_eps[1])
    newL0 = _phase(newL1, L0, ei0[1], ei0[0],
                   bw_W0[0], bw_g0[0], bw_b0[0], bw_W1[0], bw_g1[0], bw_b1[0],
                   bw_eps[0])
    newL1b = _phase(newL0, newL1, ei0[0], ei0[1],
                    fw_W0[0], fw_g0[0], fw_b0[0], fw_W1[0], fw_g1[0], fw_b1[0],
                    fw_eps[0])
    newL2 = _phase(newL1b, L2, ei1[0], ei1[1],
                   fw_W0[1], fw_g0[1], fw_b0[1], fw_W1[1], fw_g1[1], fw_b1[1],
                   fw_eps[1])
    return jnp.concatenate([newL0, newL1b, newL2], axis=0)


# speedup vs baseline: 3.9710x; 3.9710x over previous
import functools

import jax
import jax.numpy as jnp
from jax import lax
from jax.experimental import pallas as pl
from jax.experimental.pallas import tpu as pltpu
from jax.experimental.pallas import tpu_sc as plsc

NHID = 128
HALF = 64
NPER = 16384
E = 262144
NC = 2
NS = 16
CH = 128
EDGES_PER_TILE = E // NS
NCHUNK = EDGES_PER_TILE // CH
ROWS_PER_TILE = NPER // NS


def _sc_segsum_body(table, src_hbm, dst_hbm, out, sidx, didx, rows, acc, sem):
    c = lax.axis_index("c")
    s = lax.axis_index("s")

    zero = jnp.zeros((16,), jnp.float32)

    def zrow(i, carry):
        for j in range(HALF // 16):
            rows[i, pl.ds(j * 16, 16)] = zero
        return carry

    lax.fori_loop(0, CH, zrow, 0)
    for k in range(ROWS_PER_TILE // CH):
        pltpu.sync_copy(rows, acc.at[pl.ds(s * ROWS_PER_TILE + k * CH, CH)])
    plsc.subcore_barrier()

    coff = c * NPER

    def chunk(k, carry):
        base = s * EDGES_PER_TILE + k * CH
        pltpu.sync_copy(src_hbm.at[pl.ds(base, CH)], sidx)
        pltpu.sync_copy(dst_hbm.at[pl.ds(base, CH)], didx)
        for j in range(CH // 16):
            sidx[pl.ds(j * 16, 16)] = sidx[pl.ds(j * 16, 16)] + coff
        pltpu.async_copy(table.at[sidx], rows, sem).wait()
        pltpu.sync_copy(rows, acc.at[didx], add=True)
        return carry

    lax.fori_loop(0, NCHUNK, chunk, 0)
    plsc.subcore_barrier()
    pltpu.sync_copy(acc.at[pl.ds(s * ROWS_PER_TILE, ROWS_PER_TILE)],
                    out.at[pl.ds(coff + s * ROWS_PER_TILE, ROWS_PER_TILE)])


@functools.cache
def _get_sc_segsum():
    return functools.partial(
        pl.kernel,
        mesh=plsc.VectorSubcoreMesh(core_axis_name="c", subcore_axis_name="s"),
        out_type=jax.ShapeDtypeStruct((NC * NPER, HALF), jnp.float32),
        scratch_types=[
            pltpu.VMEM((CH,), jnp.int32),
            pltpu.VMEM((CH,), jnp.int32),
            pltpu.VMEM((CH, HALF), jnp.float32),
            pltpu.VMEM_SHARED((NPER, HALF), jnp.float32),
            pltpu.SemaphoreType.DMA,
        ],
        compiler_params=pltpu.CompilerParams(use_tc_tiling_on_sc=False),
    )(_sc_segsum_body)


def _sc_segsum(table, src_idx, dst_idx):
    return _get_sc_segsum()(table, src_idx, dst_idx)


def _mlp_body(a_ref, xres_ref, w0_ref, g0_ref, b0_ref, w1_ref, g1_ref,
              b1_ref, eps_ref, out_ref):
    a = a_ref[...]
    h = lax.dot_general(a, w0_ref[...], (((1,), (1,)), ((), ())),
                        preferred_element_type=jnp.float32)
    m = jnp.mean(h, axis=0, keepdims=True)
    hm = h - m
    v = jnp.mean(hm * hm, axis=0, keepdims=True)
    x1 = jnp.maximum(hm * lax.rsqrt(v + 1e-5) * g0_ref[...] + b0_ref[...], 0.0)
    h2 = lax.dot_general(x1, w1_ref[...], (((1,), (1,)), ((), ())),
                         preferred_element_type=jnp.float32)
    m2 = jnp.mean(h2, axis=0, keepdims=True)
    hm2 = h2 - m2
    v2 = jnp.mean(hm2 * hm2, axis=0, keepdims=True)
    y = jnp.maximum(hm2 * lax.rsqrt(v2 + 1e-5) * g1_ref[...] + b1_ref[...], 0.0)
    out_ref[...] = (1.0 + eps_ref[0, 0]) * xres_ref[...] + y


_mlp = pl.pallas_call(
    _mlp_body,
    out_shape=jax.ShapeDtypeStruct((NPER, NHID), jnp.float32),
    in_specs=[pl.BlockSpec(memory_space=pltpu.VMEM)] * 8
             + [pl.BlockSpec(memory_space=pltpu.SMEM)],
)


def _split(x):
    n = x.shape[0]
    return x.reshape(n, 2, HALF).transpose(1, 0, 2).reshape(2 * n, HALF)


def _merge(t):
    n = t.shape[0] // 2
    return t.reshape(2, n, HALF).transpose(1, 0, 2).reshape(n, NHID)


def _phase(x_src, x_res, src_idx, dst_idx, W0, g0, b0, W1, g1, b1, eps):
    agg = _sc_segsum(_split(x_src), src_idx, dst_idx)
    return _mlp(_merge(agg), x_res,
                W0, g0.reshape(1, NHID), b0.reshape(1, NHID),
                W1, g1.reshape(1, NHID), b1.reshape(1, NHID),
                eps.reshape(1, 1))


def kernel(xs, k_batch, bipartites, bw_W0, bw_g0, bw_b0, bw_W1, bw_g1, bw_b1,
           fw_W0, fw_g0, fw_b0, fw_W1, fw_g1, fw_b1, bw_eps, fw_eps):
    L0 = xs[0:NPER]
    L1 = xs[NPER:2 * NPER]
    L2 = xs[2 * NPER:3 * NPER]
    ei0 = bipartites[0]
    ei1 = bipartites[1]

    newL1 = _phase(L2, L1, ei1[1], ei1[0],
                   bw_W0[1], bw_g0[1], bw_b0[1], bw_W1[1], bw_g1[1], bw_b1[1],
                   bw_eps[1])
    newL0 = _phase(newL1, L0, ei0[1], ei0[0],
                   bw_W0[0], bw_g0[0], bw_b0[0], bw_W1[0], bw_g1[0], bw_b1[0],
                   bw_eps[0])
    newL1b = _phase(newL0, newL1, ei0[0], ei0[1],
                    fw_W0[0], fw_g0[0], fw_b0[0], fw_W1[0], fw_g1[0], fw_b1[0],
                    fw_eps[0])
    newL2 = _phase(newL1b, L2, ei1[0], ei1[1],
                   fw_W0[1], fw_g0[1], fw_b0[1], fw_W1[1], fw_g1[1], fw_b1[1],
                   fw_eps[1])
    return jnp.concatenate([newL0, newL1b, newL2], axis=0)


# speedup vs baseline: 8.4703x; 2.1331x over previous
import functools

import jax
import jax.numpy as jnp
from jax import lax
from jax.experimental import pallas as pl
from jax.experimental.pallas import tpu as pltpu
from jax.experimental.pallas import tpu_sc as plsc

NHID = 128
HALF = 64
NPER = 16384
E = 262144
NC = 2
NS = 16
CH = 128
EDGES_PER_TILE = E // NS
NCHUNK = EDGES_PER_TILE // CH
ROWS_PER_TILE = NPER // NS


NBUF = 4


def _sc_segsum_body(table, src_hbm, dst_hbm, out, sidx, didx, rows, acc,
                    gsem, ssem):
    c = lax.axis_index("c")
    s = lax.axis_index("s")

    zero = jnp.zeros((16,), jnp.float32)

    def zrow(i, carry):
        for j in range(HALF // 16):
            rows[0][i, pl.ds(j * 16, 16)] = zero
        return carry

    lax.fori_loop(0, CH, zrow, 0)
    for k in range(ROWS_PER_TILE // CH):
        pltpu.sync_copy(rows[0], acc.at[pl.ds(s * ROWS_PER_TILE + k * CH, CH)])

    pltpu.sync_copy(src_hbm.at[pl.ds((c * NS + s) * NCHUNK, NCHUNK)], sidx)
    pltpu.sync_copy(dst_hbm.at[pl.ds(s * NCHUNK, NCHUNK)], didx)
    plsc.subcore_barrier()

    for b in range(NBUF - 1):
        pltpu.async_copy(table.at[sidx.at[b]], rows[b], gsem[b])

    def step(k2, carry):
        for b in range(NBUF):
            k = k2 * NBUF + b
            pltpu.make_async_copy(table.at[sidx.at[0]], rows[b],
                                  gsem[b]).wait()
            pltpu.async_copy(rows[b], acc.at[didx.at[k]], ssem[b], add=True)
            b3 = (b - 1) % NBUF
            kn = k + NBUF - 1

            @pl.when(kn < NCHUNK)
            def _():
                @pl.when(k >= 1)
                def _():
                    pltpu.make_async_copy(rows[b3], acc.at[didx.at[0]],
                                          ssem[b3]).wait()
                pltpu.async_copy(table.at[sidx.at[kn]], rows[b3], gsem[b3])

        return carry

    lax.fori_loop(0, NCHUNK // NBUF, step, 0)
    for b in range(NBUF):
        pltpu.make_async_copy(rows[b], acc.at[didx.at[0]], ssem[b]).wait()
    plsc.subcore_barrier()
    pltpu.sync_copy(acc.at[pl.ds(s * ROWS_PER_TILE, ROWS_PER_TILE)],
                    out.at[pl.ds(c * NPER + s * ROWS_PER_TILE,
                                 ROWS_PER_TILE)])


@functools.cache
def _get_sc_segsum():
    return functools.partial(
        pl.kernel,
        mesh=plsc.VectorSubcoreMesh(core_axis_name="c", subcore_axis_name="s"),
        out_type=jax.ShapeDtypeStruct((NC * NPER, HALF), jnp.float32),
        scratch_types=[
            pltpu.VMEM((NCHUNK, CH), jnp.int32),
            pltpu.VMEM((NCHUNK, CH), jnp.int32),
            [pltpu.VMEM((CH, HALF), jnp.float32)] * NBUF,
            pltpu.VMEM_SHARED((NPER, HALF), jnp.float32),
            [pltpu.SemaphoreType.DMA] * NBUF,
            [pltpu.SemaphoreType.DMA] * NBUF,
        ],
        compiler_params=pltpu.CompilerParams(use_tc_tiling_on_sc=False),
    )(_sc_segsum_body)


def _sc_segsum(table, src_idx2, dst_idx2):
    return _get_sc_segsum()(table, src_idx2, dst_idx2)


def _mlp_body(a_ref, xres_ref, w0_ref, g0_ref, b0_ref, w1_ref, g1_ref,
              b1_ref, eps_ref, out_ref):
    a = a_ref[...]
    h = lax.dot_general(a, w0_ref[...], (((1,), (1,)), ((), ())),
                        preferred_element_type=jnp.float32)
    m = jnp.mean(h, axis=0, keepdims=True)
    hm = h - m
    v = jnp.mean(hm * hm, axis=0, keepdims=True)
    x1 = jnp.maximum(hm * lax.rsqrt(v + 1e-5) * g0_ref[...] + b0_ref[...], 0.0)
    h2 = lax.dot_general(x1, w1_ref[...], (((1,), (1,)), ((), ())),
                         preferred_element_type=jnp.float32)
    m2 = jnp.mean(h2, axis=0, keepdims=True)
    hm2 = h2 - m2
    v2 = jnp.mean(hm2 * hm2, axis=0, keepdims=True)
    y = jnp.maximum(hm2 * lax.rsqrt(v2 + 1e-5) * g1_ref[...] + b1_ref[...], 0.0)
    out_ref[...] = (1.0 + eps_ref[0, 0]) * xres_ref[...] + y


_mlp = pl.pallas_call(
    _mlp_body,
    out_shape=jax.ShapeDtypeStruct((NPER, NHID), jnp.float32),
    in_specs=[pl.BlockSpec(memory_space=pltpu.VMEM)] * 8
             + [pl.BlockSpec(memory_space=pltpu.SMEM)],
)


def _split(x):
    n = x.shape[0]
    return x.reshape(n, 2, HALF).transpose(1, 0, 2).reshape(2 * n, HALF)


def _merge(t):
    n = t.shape[0] // 2
    return t.reshape(2, n, HALF).transpose(1, 0, 2).reshape(n, NHID)


def _phase(x_src, x_res, src_idx, dst_idx, W0, g0, b0, W1, g1, b1, eps):
    src2 = jnp.stack([src_idx, src_idx + NPER]).reshape(NC * NS * NCHUNK, CH)
    dst2 = dst_idx.reshape(NS * NCHUNK, CH)
    agg = _sc_segsum(_split(x_src), src2, dst2)
    return _mlp(_merge(agg), x_res,
                W0, g0.reshape(1, NHID), b0.reshape(1, NHID),
                W1, g1.reshape(1, NHID), b1.reshape(1, NHID),
                eps.reshape(1, 1))


def kernel(xs, k_batch, bipartites, bw_W0, bw_g0, bw_b0, bw_W1, bw_g1, bw_b1,
           fw_W0, fw_g0, fw_b0, fw_W1, fw_g1, fw_b1, bw_eps, fw_eps):
    L0 = xs[0:NPER]
    L1 = xs[NPER:2 * NPER]
    L2 = xs[2 * NPER:3 * NPER]
    ei0 = bipartites[0]
    ei1 = bipartites[1]

    newL1 = _phase(L2, L1, ei1[1], ei1[0],
                   bw_W0[1], bw_g0[1], bw_b0[1], bw_W1[1], bw_g1[1], bw_b1[1],
                   bw_eps[1])
    newL0 = _phase(newL1, L0, ei0[1], ei0[0],
                   bw_W0[0], bw_g0[0], bw_b0[0], bw_W1[0], bw_g1[0], bw_b1[0],
                   bw_eps[0])
    newL1b = _phase(newL0, newL1, ei0[0], ei0[1],
                    fw_W0[0], fw_g0[0], fw_b0[0], fw_W1[0], fw_g1[0], fw_b1[0],
                    fw_eps[0])
    newL2 = _phase(newL1b, L2, ei1[0], ei1[1],
                   fw_W0[1], fw_g0[1], fw_b0[1], fw_W1[1], fw_g1[1], fw_b1[1],
                   fw_eps[1])
    return jnp.concatenate([newL0, newL1b, newL2], axis=0)


# speedup vs baseline: 9.6739x; 1.1421x over previous
import functools

import jax
import jax.numpy as jnp
from jax import lax
from jax.experimental import pallas as pl
from jax.experimental.pallas import tpu as pltpu
from jax.experimental.pallas import tpu_sc as plsc

NHID = 128
HALF = 64
NPER = 16384
E = 262144
NC = 2
NS = 16
CH = 128
EDGES_PER_TILE = E // NS
NCHUNK = EDGES_PER_TILE // CH
ROWS_PER_TILE = NPER // NS


NBUF = 4


def _sc_segsum_body(table, src_hbm, dst_hbm, out, sidx, didx, rows, acc,
                    gsem, ssem):
    c = lax.axis_index("c")
    s = lax.axis_index("s")

    zero = jnp.zeros((16,), jnp.float32)

    def zrow(i, carry):
        for j in range(HALF // 16):
            rows[0][i, pl.ds(j * 16, 16)] = zero
        return carry

    lax.fori_loop(0, CH, zrow, 0)
    for k in range(ROWS_PER_TILE // CH):
        pltpu.sync_copy(rows[0], acc.at[pl.ds(s * ROWS_PER_TILE + k * CH, CH)])

    pltpu.sync_copy(src_hbm.at[pl.ds((c * NS + s) * NCHUNK, NCHUNK)], sidx)
    pltpu.sync_copy(dst_hbm.at[pl.ds(s * NCHUNK, NCHUNK)], didx)
    plsc.subcore_barrier()

    for b in range(NBUF - 1):
        pltpu.async_copy(table.at[sidx.at[b]], rows[b], gsem[b])

    def step(k2, carry):
        for b in range(NBUF):
            k = k2 * NBUF + b
            pltpu.make_async_copy(table.at[sidx.at[0]], rows[b],
                                  gsem[b]).wait()
            pltpu.async_copy(rows[b], acc.at[didx.at[k]], ssem[b], add=True)
            b3 = (b - 1) % NBUF
            kn = k + NBUF - 1

            @pl.when(kn < NCHUNK)
            def _():
                @pl.when(k >= 1)
                def _():
                    pltpu.make_async_copy(rows[b3], acc.at[didx.at[0]],
                                          ssem[b3]).wait()
                pltpu.async_copy(table.at[sidx.at[kn]], rows[b3], gsem[b3])

        return carry

    lax.fori_loop(0, NCHUNK // NBUF, step, 0)
    for b in range(NBUF):
        pltpu.make_async_copy(rows[b], acc.at[didx.at[0]], ssem[b]).wait()
    plsc.subcore_barrier()
    pltpu.sync_copy(acc.at[pl.ds(s * ROWS_PER_TILE, ROWS_PER_TILE)],
                    out.at[pl.ds(c * NPER + s * ROWS_PER_TILE,
                                 ROWS_PER_TILE)])


@functools.cache
def _get_sc_segsum():
    return functools.partial(
        pl.kernel,
        mesh=plsc.VectorSubcoreMesh(core_axis_name="c", subcore_axis_name="s"),
        out_type=jax.ShapeDtypeStruct((NC * NPER, HALF), jnp.float32),
        scratch_types=[
            pltpu.VMEM((NCHUNK, CH), jnp.int32),
            pltpu.VMEM((NCHUNK, CH), jnp.int32),
            [pltpu.VMEM((CH, HALF), jnp.float32)] * NBUF,
            pltpu.VMEM_SHARED((NPER, HALF), jnp.float32),
            [pltpu.SemaphoreType.DMA] * NBUF,
            [pltpu.SemaphoreType.DMA] * NBUF,
        ],
        compiler_params=pltpu.CompilerParams(use_tc_tiling_on_sc=False),
    )(_sc_segsum_body)


def _sc_segsum(table, src_idx2, dst_idx2):
    return _get_sc_segsum()(table, src_idx2, dst_idx2)


def _mlp_body(a_ref, xres_ref, w0_ref, g0_ref, b0_ref, w1_ref, g1_ref,
              b1_ref, eps_ref, split_ref, full_ref):
    a0 = a_ref[0:NPER, :]
    a1 = a_ref[NPER:2 * NPER, :]
    w0 = w0_ref[...]
    h = (lax.dot_general(a0, w0[:, 0:HALF], (((1,), (1,)), ((), ())),
                         preferred_element_type=jnp.float32)
         + lax.dot_general(a1, w0[:, HALF:NHID], (((1,), (1,)), ((), ())),
                           preferred_element_type=jnp.float32))
    m = jnp.mean(h, axis=0, keepdims=True)
    hm = h - m
    v = jnp.mean(hm * hm, axis=0, keepdims=True)
    x1 = jnp.maximum(hm * lax.rsqrt(v + 1e-5) * g0_ref[...] + b0_ref[...], 0.0)
    h2 = lax.dot_general(x1, w1_ref[...], (((1,), (1,)), ((), ())),
                         preferred_element_type=jnp.float32)
    m2 = jnp.mean(h2, axis=0, keepdims=True)
    hm2 = h2 - m2
    v2 = jnp.mean(hm2 * hm2, axis=0, keepdims=True)
    y = jnp.maximum(hm2 * lax.rsqrt(v2 + 1e-5) * g1_ref[...] + b1_ref[...], 0.0)
    out = (1.0 + eps_ref[0, 0]) * xres_ref[...] + y
    full_ref[...] = out
    split_ref[0:NPER, :] = out[:, 0:HALF]
    split_ref[NPER:2 * NPER, :] = out[:, HALF:NHID]


_mlp = pl.pallas_call(
    _mlp_body,
    out_shape=(jax.ShapeDtypeStruct((2 * NPER, HALF), jnp.float32),
               jax.ShapeDtypeStruct((NPER, NHID), jnp.float32)),
    in_specs=[pl.BlockSpec(memory_space=pltpu.VMEM)] * 8
             + [pl.BlockSpec(memory_space=pltpu.SMEM)],
)


def _split(x):
    n = x.shape[0]
    return x.reshape(n, 2, HALF).transpose(1, 0, 2).reshape(2 * n, HALF)


def _phase(tab_split, x_res_full, src_idx, dst_idx, W0, g0, b0, W1, g1, b1,
           eps):
    src2 = jnp.stack([src_idx, src_idx + NPER]).reshape(NC * NS * NCHUNK, CH)
    dst2 = dst_idx.reshape(NS * NCHUNK, CH)
    agg = _sc_segsum(tab_split, src2, dst2)
    return _mlp(agg, x_res_full,
                W0, g0.reshape(1, NHID), b0.reshape(1, NHID),
                W1, g1.reshape(1, NHID), b1.reshape(1, NHID),
                eps.reshape(1, 1))


def kernel(xs, k_batch, bipartites, bw_W0, bw_g0, bw_b0, bw_W1, bw_g1, bw_b1,
           fw_W0, fw_g0, fw_b0, fw_W1, fw_g1, fw_b1, bw_eps, fw_eps):
    L0 = xs[0:NPER]
    L1 = xs[NPER:2 * NPER]
    L2 = xs[2 * NPER:3 * NPER]
    ei0 = bipartites[0]
    ei1 = bipartites[1]

    nl1_s, nl1_f = _phase(_split(L2), L1, ei1[1], ei1[0],
                          bw_W0[1], bw_g0[1], bw_b0[1], bw_W1[1], bw_g1[1],
                          bw_b1[1], bw_eps[1])
    nl0_s, nl0_f = _phase(nl1_s, L0, ei0[1], ei0[0],
                          bw_W0[0], bw_g0[0], bw_b0[0], bw_W1[0], bw_g1[0],
                          bw_b1[0], bw_eps[0])
    nl1b_s, nl1b_f = _phase(nl0_s, nl1_f, ei0[0], ei0[1],
                            fw_W0[0], fw_g0[0], fw_b0[0], fw_W1[0], fw_g1[0],
                            fw_b1[0], fw_eps[0])
    _, nl2_f = _phase(nl1b_s, L2, ei1[0], ei1[1],
                      fw_W0[1], fw_g0[1], fw_b0[1], fw_W1[1], fw_g1[1],
                      fw_b1[1], fw_eps[1])
    return jnp.concatenate([nl0_f, nl1b_f, nl2_f], axis=0)


# speedup vs baseline: 10.9430x; 1.1312x over previous
import functools

import jax
import jax.numpy as jnp
from jax import lax
from jax.experimental import pallas as pl
from jax.experimental.pallas import tpu as pltpu
from jax.experimental.pallas import tpu_sc as plsc

NHID = 128
HALF = 64
NPER = 16384
E = 262144
NC = 2
NS = 16
CH = 128
EDGES_PER_TILE = E // NS
NCHUNK = EDGES_PER_TILE // CH
ROWS_PER_TILE = NPER // NS


NBUF = 4


def _sc_segsum_body(table, src_hbm, dst_hbm, out, sidx, didx, rows, acc,
                    gsem, ssem):
    c = lax.axis_index("c")
    s = lax.axis_index("s")

    zero = jnp.zeros((32,), jnp.bfloat16)

    def zrow(i, carry):
        for j in range(HALF // 32):
            rows[0][i, pl.ds(j * 32, 32)] = zero
        return carry

    lax.fori_loop(0, CH, zrow, 0)
    for k in range(ROWS_PER_TILE // CH):
        pltpu.sync_copy(rows[0], acc.at[pl.ds(s * ROWS_PER_TILE + k * CH, CH)])

    pltpu.sync_copy(src_hbm.at[pl.ds((c * NS + s) * NCHUNK, NCHUNK)], sidx)
    pltpu.sync_copy(dst_hbm.at[pl.ds(s * NCHUNK, NCHUNK)], didx)
    plsc.subcore_barrier()

    for b in range(NBUF - 1):
        pltpu.async_copy(table.at[sidx.at[b]], rows[b], gsem[b])

    def step(k2, carry):
        for b in range(NBUF):
            k = k2 * NBUF + b
            pltpu.make_async_copy(table.at[sidx.at[0]], rows[b],
                                  gsem[b]).wait()
            pltpu.async_copy(rows[b], acc.at[didx.at[k]], ssem[b], add=True)
            b3 = (b - 1) % NBUF
            kn = k + NBUF - 1

            @pl.when(kn < NCHUNK)
            def _():
                @pl.when(k >= 1)
                def _():
                    pltpu.make_async_copy(rows[b3], acc.at[didx.at[0]],
                                          ssem[b3]).wait()
                pltpu.async_copy(table.at[sidx.at[kn]], rows[b3], gsem[b3])

        return carry

    lax.fori_loop(0, NCHUNK // NBUF, step, 0)
    for b in range(NBUF):
        pltpu.make_async_copy(rows[b], acc.at[didx.at[0]], ssem[b]).wait()
    plsc.subcore_barrier()
    pltpu.sync_copy(acc.at[pl.ds(s * ROWS_PER_TILE, ROWS_PER_TILE)],
                    out.at[pl.ds(c * NPER + s * ROWS_PER_TILE,
                                 ROWS_PER_TILE)])


@functools.cache
def _get_sc_segsum():
    return functools.partial(
        pl.kernel,
        mesh=plsc.VectorSubcoreMesh(core_axis_name="c", subcore_axis_name="s"),
        out_type=jax.ShapeDtypeStruct((NC * NPER, HALF), jnp.bfloat16),
        scratch_types=[
            pltpu.VMEM((NCHUNK, CH), jnp.int32),
            pltpu.VMEM((NCHUNK, CH), jnp.int32),
            [pltpu.VMEM((CH, HALF), jnp.bfloat16)] * NBUF,
            pltpu.VMEM_SHARED((NPER, HALF), jnp.bfloat16),
            [pltpu.SemaphoreType.DMA] * NBUF,
            [pltpu.SemaphoreType.DMA] * NBUF,
        ],
        compiler_params=pltpu.CompilerParams(use_tc_tiling_on_sc=False),
    )(_sc_segsum_body)


def _sc_segsum(table, src_idx2, dst_idx2):
    return _get_sc_segsum()(table, src_idx2, dst_idx2)


def _mlp_body(a_ref, xres_ref, w0_ref, g0_ref, b0_ref, w1_ref, g1_ref,
              b1_ref, eps_ref, split_ref, full_ref):
    a0 = a_ref[0:NPER, :].astype(jnp.float32)
    a1 = a_ref[NPER:2 * NPER, :].astype(jnp.float32)
    w0 = w0_ref[...]
    h = (lax.dot_general(a0, w0[:, 0:HALF], (((1,), (1,)), ((), ())),
                         preferred_element_type=jnp.float32)
         + lax.dot_general(a1, w0[:, HALF:NHID], (((1,), (1,)), ((), ())),
                           preferred_element_type=jnp.float32))
    m = jnp.mean(h, axis=0, keepdims=True)
    hm = h - m
    v = jnp.mean(hm * hm, axis=0, keepdims=True)
    x1 = jnp.maximum(hm * lax.rsqrt(v + 1e-5) * g0_ref[...] + b0_ref[...], 0.0)
    h2 = lax.dot_general(x1, w1_ref[...], (((1,), (1,)), ((), ())),
                         preferred_element_type=jnp.float32)
    m2 = jnp.mean(h2, axis=0, keepdims=True)
    hm2 = h2 - m2
    v2 = jnp.mean(hm2 * hm2, axis=0, keepdims=True)
    y = jnp.maximum(hm2 * lax.rsqrt(v2 + 1e-5) * g1_ref[...] + b1_ref[...], 0.0)
    out = (1.0 + eps_ref[0, 0]) * xres_ref[...] + y
    full_ref[...] = out
    outh = out.astype(jnp.bfloat16)
    split_ref[0:NPER, :] = outh[:, 0:HALF]
    split_ref[NPER:2 * NPER, :] = outh[:, HALF:NHID]


_mlp = pl.pallas_call(
    _mlp_body,
    out_shape=(jax.ShapeDtypeStruct((2 * NPER, HALF), jnp.bfloat16),
               jax.ShapeDtypeStruct((NPER, NHID), jnp.float32)),
    in_specs=[pl.BlockSpec(memory_space=pltpu.VMEM)] * 8
             + [pl.BlockSpec(memory_space=pltpu.SMEM)],
)


def _split(x):
    n = x.shape[0]
    t = x.reshape(n, 2, HALF).transpose(1, 0, 2).reshape(2 * n, HALF)
    return t.astype(jnp.bfloat16)


def _phase(tab_split, x_res_full, src_idx, dst_idx, W0, g0, b0, W1, g1, b1,
           eps):
    src2 = jnp.stack([src_idx, src_idx + NPER]).reshape(NC * NS * NCHUNK, CH)
    dst2 = dst_idx.reshape(NS * NCHUNK, CH)
    agg = _sc_segsum(tab_split, src2, dst2)
    return _mlp(agg, x_res_full,
                W0, g0.reshape(1, NHID), b0.reshape(1, NHID),
                W1, g1.reshape(1, NHID), b1.reshape(1, NHID),
                eps.reshape(1, 1))


def kernel(xs, k_batch, bipartites, bw_W0, bw_g0, bw_b0, bw_W1, bw_g1, bw_b1,
           fw_W0, fw_g0, fw_b0, fw_W1, fw_g1, fw_b1, bw_eps, fw_eps):
    L0 = xs[0:NPER]
    L1 = xs[NPER:2 * NPER]
    L2 = xs[2 * NPER:3 * NPER]
    ei0 = bipartites[0]
    ei1 = bipartites[1]

    nl1_s, nl1_f = _phase(_split(L2), L1, ei1[1], ei1[0],
                          bw_W0[1], bw_g0[1], bw_b0[1], bw_W1[1], bw_g1[1],
                          bw_b1[1], bw_eps[1])
    nl0_s, nl0_f = _phase(nl1_s, L0, ei0[1], ei0[0],
                          bw_W0[0], bw_g0[0], bw_b0[0], bw_W1[0], bw_g1[0],
                          bw_b1[0], bw_eps[0])
    nl1b_s, nl1b_f = _phase(nl0_s, nl1_f, ei0[0], ei0[1],
                            fw_W0[0], fw_g0[0], fw_b0[0], fw_W1[0], fw_g1[0],
                            fw_b1[0], fw_eps[0])
    _, nl2_f = _phase(nl1b_s, L2, ei1[0], ei1[1],
                      fw_W0[1], fw_g0[1], fw_b0[1], fw_W1[1], fw_g1[1],
                      fw_b1[1], fw_eps[1])
    return jnp.concatenate([nl0_f, nl1b_f, nl2_f], axis=0)
